# baseline (speedup 1.0000x reference)
import numpy as np
import jax
import jax.numpy as jnp
from jax import lax
from jax.experimental import pallas as pl
from jax.experimental.pallas import tpu as pltpu
from jax.experimental.pallas import tpu_sc as plsc

TRAIN = 140
STEPS = 38
T = TRAIN + STEPS
XW = T * 5
XIW = T * 11

X_OFF = 0
XIF_OFF = 896
XD_OFF = 2864
PB = 2880
_EMB = [(2880, 5, 2), (2915, 5, 4), (2950, 2, 5), (2964, 10, 6), (3034, 5, 7)]
OH7 = 3072
I38 = 3128
PAL_LEN = 4576
ENC_W = 48 * TRAIN
DEC_W = 84 * STEPS
DEC_WP = 3200
NW = 32
UNR = 10
GI_BASE = XIF_OFF


def _pack(sb, mu, gi, isf):
    return sb + (mu << 13) + (gi << 17) + (isf << 31)


def _build_maps():
    def emb_entries(t):
        out = []
        for base, dim, col in _EMB:
            for k in range(dim):
                out.append(_pack(base + k, dim, GI_BASE + t * 11 + col, 0))
        return out

    enc = []
    for t in range(TRAIN):
        rows = [_pack(X_OFF + t * 5 + c, 0, GI_BASE, 0) for c in range(5)]
        rows += emb_entries(t)
        rows += [_pack(XD_OFF + k, 0, GI_BASE, 0) for k in range(5)]
        rows.append(_pack(0, 0, GI_BASE + t * 11 + 0, 1))
        rows += [_pack(0, 0, GI_BASE + t * 11 + k, 1) for k in (8, 9, 10)]
        rows += [_pack(OH7 + k, 7, GI_BASE + t * 11 + 1, 0) for k in range(7)]
        enc += rows
    dec = []
    for s in range(STEPS):
        t = TRAIN + s
        rows = [_pack(X_OFF + t * 5 + 0, 0, GI_BASE, 0)]
        rows += emb_entries(t)
        rows += [_pack(X_OFF + t * 5 + k, 0, GI_BASE, 0) for k in (2, 3, 4)]
        rows += [_pack(XD_OFF + k, 0, GI_BASE, 0) for k in range(5)]
        rows += [_pack(0, 0, GI_BASE + t * 11 + k, 1) for k in (9, 10)]
        rows.append(_pack(0, 0, GI_BASE + t * 11 + 0, 1))
        rows += [_pack(I38 + s * 38 + k, 0, GI_BASE, 0) for k in range(38)]
        rows += [_pack(OH7 + k, 7, GI_BASE + t * 11 + 1, 0) for k in range(7)]
        dec += rows
    dec += [_pack(0, 0, GI_BASE, 0)] * (DEC_WP - DEC_W)
    e = (np.array(enc, np.int64) & 0xFFFFFFFF).astype(np.uint32).view(np.int32)
    d = (np.array(dec, np.int64) & 0xFFFFFFFF).astype(np.uint32).view(np.int32)
    return e, d


def _renorm(W, m):
    n = jnp.sqrt(jnp.sum(W * W, axis=1, keepdims=True))
    return W * jnp.minimum(1.0, m / jnp.maximum(n, 1e-7))


def _static_pal(day_W, genre_W, pref_W, area_W, muni_W):
    parts = [
        _renorm(day_W, 5.0)[:7].reshape(-1),
        _renorm(genre_W, 5.0)[:7].reshape(-1),
        _renorm(pref_W, 2.0)[:7].reshape(-1),
        _renorm(area_W, 10.0)[:7].reshape(-1),
        _renorm(muni_W, 5.0)[:7].reshape(-1),
        jnp.zeros(3, jnp.float32),
        jnp.eye(7, dtype=jnp.float32).reshape(-1),
        jnp.zeros(7, jnp.float32),
        jnp.eye(38, dtype=jnp.float32).reshape(-1),
        jnp.zeros(4, jnp.float32),
    ]
    return jnp.concatenate(parts)


def _sc_body(pb_hbm, spal_hbm, pme_h, pmd_h,
             enc_hbm, dec_hbm,
             pal, pme, pmd, encv, decv):
    nb = pb_hbm.shape[0] // PB // NW
    wid = lax.axis_index("s") * 2 + lax.axis_index("c")
    b0 = wid * nb
    pltpu.sync_copy(spal_hbm, pal.at[pl.ds(PB, PAL_LEN - PB)])
    pltpu.sync_copy(pme_h, pme)
    pltpu.sync_copy(pmd_h, pmd)

    def gather_blocks(n_v, pm, outv):
        def blk(i, c):
            for v in range(UNR):
                sl = pl.ds((i * UNR + v) * 16, 16)
                p = pm[sl]
                sb = p & 0x1FFF
                mu = (p >> 13) & 0xF
                gi = (p >> 17) & 0x1FFF
                g = plsc.load_gather(pal, [gi])
                val = plsc.load_gather(pal, [sb + mu * g.astype(jnp.int32)])
                outv[sl] = jnp.where(p < 0, g, val)
            return c
        lax.fori_loop(0, n_v // UNR // 16, blk, 0)

    def per_b(i, carry):
        b = b0 + i
        pltpu.sync_copy(pb_hbm.at[pl.ds(b * PB, PB)], pal.at[pl.ds(0, PB)])
        gather_blocks(ENC_W, pme, encv)
        gather_blocks(DEC_WP, pmd, decv)
        pltpu.sync_copy(encv, enc_hbm.at[pl.ds(b * ENC_W, ENC_W)])
        pltpu.sync_copy(decv.at[pl.ds(0, DEC_W)],
                        dec_hbm.at[pl.ds(b * DEC_W, DEC_W)])
        return carry

    lax.fori_loop(0, nb, per_b, 0)


def kernel(x, x_d, day_W, genre_W, pref_W, area_W, muni_W, x_i):
    B = x.shape[0]
    pb = jnp.concatenate([
        x.reshape(B, T * 5),
        jnp.zeros((B, XIF_OFF - T * 5), jnp.float32),
        x_i.reshape(B, T * 11).astype(jnp.float32),
        jnp.zeros((B, XD_OFF - XIF_OFF - T * 11), jnp.float32),
        x_d,
        jnp.zeros((B, PB - XD_OFF - 5), jnp.float32),
    ], axis=1).reshape(-1)
    spal = _static_pal(day_W, genre_W, pref_W, area_W, muni_W)
    pme_np, pmd_np = _build_maps()
    pme, pmd = jnp.asarray(pme_np), jnp.asarray(pmd_np)

    mesh = plsc.VectorSubcoreMesh(core_axis_name="c", subcore_axis_name="s")
    run = pl.kernel(
        _sc_body,
        mesh=mesh,
        compiler_params=pltpu.CompilerParams(needs_layout_passes=False),
        out_type=[jax.ShapeDtypeStruct((B * ENC_W,), jnp.float32),
                  jax.ShapeDtypeStruct((B * DEC_W,), jnp.float32)],
        scratch_types=[
            pltpu.VMEM((PAL_LEN,), jnp.float32),
            pltpu.VMEM((ENC_W,), jnp.int32),
            pltpu.VMEM((DEC_WP,), jnp.int32),
            pltpu.VMEM((ENC_W,), jnp.float32),
            pltpu.VMEM((DEC_WP,), jnp.float32),
        ],
    )
    enc, dec = run(pb, spal, pme, pmd)
    return (enc.reshape(B, TRAIN, 48), dec.reshape(B, STEPS, 84))


# speedup vs baseline: 2.0801x; 2.0801x over previous
import numpy as np
import jax
import jax.numpy as jnp
from jax import lax
from jax.experimental import pallas as pl
from jax.experimental.pallas import tpu as pltpu
from jax.experimental.pallas import tpu_sc as plsc

TRAIN = 140
STEPS = 38
T = TRAIN + STEPS
XW = T * 5
XIW = T * 11

X_OFF = 0
XIF_OFF = 896
XD_OFF = 2864
PB = 2880
_EMB = [(2880, 5, 2), (2915, 5, 4), (2950, 2, 5), (2964, 10, 6), (3034, 5, 7)]
OH7 = 3072
I38 = 3128
PAL_LEN = 4576
ENC_W = 48 * TRAIN
DEC_W = 84 * STEPS
DEC_WP = 3200
NW = 32
UNR = 10
GI_BASE = XIF_OFF


def _pack(sb, mu, gi, isf):
    return sb + (mu << 13) + (gi << 17) + (isf << 31)


def _build_maps():
    def emb_entries(t):
        out = []
        for base, dim, col in _EMB:
            for k in range(dim):
                out.append(_pack(base + k, dim, GI_BASE + t * 11 + col, 0))
        return out

    enc = []
    for t in range(TRAIN):
        rows = [_pack(X_OFF + t * 5 + c, 0, GI_BASE, 0) for c in range(5)]
        rows += emb_entries(t)
        rows += [_pack(XD_OFF + k, 0, GI_BASE, 0) for k in range(5)]
        rows.append(_pack(0, 0, GI_BASE + t * 11 + 0, 1))
        rows += [_pack(0, 0, GI_BASE + t * 11 + k, 1) for k in (8, 9, 10)]
        rows += [_pack(OH7 + k, 7, GI_BASE + t * 11 + 1, 0) for k in range(7)]
        enc += rows
    dec = []
    for s in range(STEPS):
        t = TRAIN + s
        rows = [_pack(X_OFF + t * 5 + 0, 0, GI_BASE, 0)]
        rows += emb_entries(t)
        rows += [_pack(X_OFF + t * 5 + k, 0, GI_BASE, 0) for k in (2, 3, 4)]
        rows += [_pack(XD_OFF + k, 0, GI_BASE, 0) for k in range(5)]
        rows += [_pack(0, 0, GI_BASE + t * 11 + k, 1) for k in (9, 10)]
        rows.append(_pack(0, 0, GI_BASE + t * 11 + 0, 1))
        rows += [_pack(I38 + s * 38 + k, 0, GI_BASE, 0) for k in range(38)]
        rows += [_pack(OH7 + k, 7, GI_BASE + t * 11 + 1, 0) for k in range(7)]
        dec += rows
    dec += [_pack(0, 0, GI_BASE, 0)] * (DEC_WP - DEC_W)
    e = (np.array(enc, np.int64) & 0xFFFFFFFF).astype(np.uint32).view(np.int32)
    d = (np.array(dec, np.int64) & 0xFFFFFFFF).astype(np.uint32).view(np.int32)
    return e, d


def _renorm(W, m):
    n = jnp.sqrt(jnp.sum(W * W, axis=1, keepdims=True))
    return W * jnp.minimum(1.0, m / jnp.maximum(n, 1e-7))


def _static_pal(day_W, genre_W, pref_W, area_W, muni_W):
    parts = [
        _renorm(day_W, 5.0)[:7].reshape(-1),
        _renorm(genre_W, 5.0)[:7].reshape(-1),
        _renorm(pref_W, 2.0)[:7].reshape(-1),
        _renorm(area_W, 10.0)[:7].reshape(-1),
        _renorm(muni_W, 5.0)[:7].reshape(-1),
        jnp.zeros(3, jnp.float32),
        jnp.eye(7, dtype=jnp.float32).reshape(-1),
        jnp.zeros(7, jnp.float32),
        jnp.eye(38, dtype=jnp.float32).reshape(-1),
        jnp.zeros(4, jnp.float32),
    ]
    return jnp.concatenate(parts)


def _sc_body(pb_hbm, spal_hbm, pme_h, pmd_h,
             enc_hbm, dec_hbm,
             pal, pme, pmd, encv, decv):
    nb = pb_hbm.shape[0] // PB // NW
    wid = lax.axis_index("s") * 2 + lax.axis_index("c")
    b0 = wid * nb
    pltpu.sync_copy(spal_hbm, pal.at[pl.ds(PB, PAL_LEN - PB)])
    pltpu.sync_copy(pme_h, pme)
    pltpu.sync_copy(pmd_h, pmd)

    def gather_blocks(n_v, pm, outv):
        @plsc.parallel_loop(0, n_v // 16, 1, unroll=UNR)
        def _blk(j):
            sl = pl.ds(j * 16, 16)
            p = pm[sl]
            sb = p & 0x1FFF
            mu = (p >> 13) & 0xF
            gi = (p >> 17) & 0x1FFF
            g = plsc.load_gather(pal, [gi])
            val = plsc.load_gather(pal, [sb + mu * g.astype(jnp.int32)])
            outv[sl] = jnp.where(p < 0, g, val)

    def per_b(i, carry):
        b = b0 + i
        pltpu.sync_copy(pb_hbm.at[pl.ds(b * PB, PB)], pal.at[pl.ds(0, PB)])
        gather_blocks(ENC_W, pme, encv)
        gather_blocks(DEC_WP, pmd, decv)
        pltpu.sync_copy(encv, enc_hbm.at[pl.ds(b * ENC_W, ENC_W)])
        pltpu.sync_copy(decv.at[pl.ds(0, DEC_W)],
                        dec_hbm.at[pl.ds(b * DEC_W, DEC_W)])
        return carry

    lax.fori_loop(0, nb, per_b, 0)


def kernel(x, x_d, day_W, genre_W, pref_W, area_W, muni_W, x_i):
    B = x.shape[0]
    pb = jnp.concatenate([
        x.reshape(B, T * 5),
        jnp.zeros((B, XIF_OFF - T * 5), jnp.float32),
        x_i.reshape(B, T * 11).astype(jnp.float32),
        jnp.zeros((B, XD_OFF - XIF_OFF - T * 11), jnp.float32),
        x_d,
        jnp.zeros((B, PB - XD_OFF - 5), jnp.float32),
    ], axis=1).reshape(-1)
    spal = _static_pal(day_W, genre_W, pref_W, area_W, muni_W)
    pme_np, pmd_np = _build_maps()
    pme, pmd = jnp.asarray(pme_np), jnp.asarray(pmd_np)

    mesh = plsc.VectorSubcoreMesh(core_axis_name="c", subcore_axis_name="s")
    run = pl.kernel(
        _sc_body,
        mesh=mesh,
        compiler_params=pltpu.CompilerParams(needs_layout_passes=False),
        out_type=[jax.ShapeDtypeStruct((B * ENC_W,), jnp.float32),
                  jax.ShapeDtypeStruct((B * DEC_W,), jnp.float32)],
        scratch_types=[
            pltpu.VMEM((PAL_LEN,), jnp.float32),
            pltpu.VMEM((ENC_W,), jnp.int32),
            pltpu.VMEM((DEC_WP,), jnp.int32),
            pltpu.VMEM((ENC_W,), jnp.float32),
            pltpu.VMEM((DEC_WP,), jnp.float32),
        ],
    )
    enc, dec = run(pb, spal, pme, pmd)
    return (enc.reshape(B, TRAIN, 48), dec.reshape(B, STEPS, 84))
